# Initial kernel scaffold; baseline (speedup 1.0000x reference)
import jax
import jax.numpy as jnp
from jax.experimental import pallas as pl


def kernel(x, edge_index, edge_weight, W, b):
    raise NotImplementedError("write your pallas kernel here")


# speedup vs baseline: 4.0528x; 4.0528x over previous
import functools

import jax
import jax.numpy as jnp
from jax import lax
from jax.experimental import pallas as pl
from jax.experimental.pallas import tpu as pltpu
from jax.experimental.pallas import tpu_sc as plsc

N = 10000
E = 320000
D = 128

NC = 2
NS = 16
NW = NC * NS
EPW = E // NW
CH = 80
NCHUNK = EPW // CH
NP = 10240
RPT = NP // NS
ZR = 128


def _sc_aggregate(col, row, w, x):
    mesh = plsc.VectorSubcoreMesh(core_axis_name="c", subcore_axis_name="s")

    @functools.partial(
        pl.kernel,
        mesh=mesh,
        out_type=jax.ShapeDtypeStruct((NC, NP, D), jnp.float32),
        scratch_types=[
            pltpu.VMEM((CH,), jnp.int32),
            pltpu.VMEM((CH,), jnp.int32),
            pltpu.VMEM((CH,), jnp.float32),
            pltpu.VMEM((CH, D), jnp.float32),
            pltpu.VMEM((ZR, D), jnp.float32),
            pltpu.VMEM_SHARED((NP, D), jnp.float32),
            pltpu.SemaphoreType.DMA,
        ],
    )
    def agg(col_hbm, row_hbm, w_hbm, x_hbm, out_hbm,
            colv, rowv, wv, rowsv, zbuf, acc, sem):
        c = lax.axis_index("c")
        s = lax.axis_index("s")
        wid = s * NC + c

        zero16 = jnp.zeros((16,), jnp.float32)

        def zrow(i, _):
            for j in range(D // 16):
                zbuf[i, pl.ds(j * 16, 16)] = zero16
            return 0

        lax.fori_loop(0, ZR, zrow, 0)
        for q in range(RPT // ZR):
            pltpu.sync_copy(zbuf, acc.at[pl.ds(s * RPT + q * ZR, ZR)])
        plsc.subcore_barrier()

        base = wid * EPW

        def chunk(i, _):
            off = base + i * CH
            pltpu.sync_copy(col_hbm.at[pl.ds(off, CH)], colv)
            pltpu.sync_copy(row_hbm.at[pl.ds(off, CH)], rowv)
            pltpu.sync_copy(w_hbm.at[pl.ds(off, CH)], wv)
            pltpu.async_copy(x_hbm.at[colv], rowsv, sem).wait()

            def group(g, _):
                wchunk = wv[pl.ds(g * 16, 16)]
                for t in range(16):
                    wvec = jnp.full((16,), wchunk[t], jnp.float32)
                    e = g * 16 + t
                    for j in range(D // 16):
                        sl = pl.ds(j * 16, 16)
                        rowsv[e, sl] = rowsv[e, sl] * wvec
                return 0

            lax.fori_loop(0, CH // 16, group, 0)
            pltpu.sync_copy(rowsv, acc.at[rowv], add=True)
            return 0

        lax.fori_loop(0, NCHUNK, chunk, 0)
        plsc.subcore_barrier()

        pltpu.sync_copy(acc.at[pl.ds(s * RPT, RPT)],
                        out_hbm.at[c, pl.ds(s * RPT, RPT)])

    return agg(col, row, w, x)


BLK = 400


def _tc_finish(p0, p1, W, b2d):
    def body(p0_ref, p1_ref, w_ref, b_ref, o_ref):
        agg = p0_ref[...] + p1_ref[...]
        o_ref[...] = lax.dot_general(
            agg, w_ref[...], (((1,), (1,)), ((), ())),
            preferred_element_type=jnp.float32) + b_ref[...]

    return pl.pallas_call(
        body,
        grid=(N // BLK,),
        in_specs=[
            pl.BlockSpec((BLK, D), lambda i: (i, 0)),
            pl.BlockSpec((BLK, D), lambda i: (i, 0)),
            pl.BlockSpec((D, D), lambda i: (0, 0)),
            pl.BlockSpec((1, D), lambda i: (0, 0)),
        ],
        out_specs=pl.BlockSpec((BLK, D), lambda i: (i, 0)),
        out_shape=jax.ShapeDtypeStruct((N, D), jnp.float32),
    )(p0, p1, W, b2d)


def kernel(x, edge_index, edge_weight, W, b):
    row = edge_index[0].astype(jnp.int32)
    col = edge_index[1].astype(jnp.int32)
    partials = _sc_aggregate(col, row, edge_weight, x)
    return _tc_finish(partials[0], partials[1], W, b.reshape(1, D))
